# baseline (speedup 1.0000x reference)
import jax
import jax.numpy as jnp
from jax import lax
from jax.experimental import pallas as pl
from jax.experimental.pallas import tpu as pltpu
from jax.experimental.pallas import tpu_sc as plsc

_M = 128
_BN = 3584
_NPAD = 100352
_NW = 32
_W = _NPAD // _NW
_PAIR = 2
_L = 16
_MR = _M * _L


def _sc_core(x_hbm, y_hbm, z_hbm, cx_hbm, cy_hbm, cz_hbm,
             hx_hbm, hy_hbm, hz_hbm,
             mind_hbm, nsel_hbm,
             xv, yv, zv, mv, nv, cxv, cyv, czv, n2v, hxv, hyv, hzv):
    wid = lax.axis_index("s") * 2 + lax.axis_index("c")
    base = wid * _W
    pltpu.sync_copy(x_hbm.at[pl.ds(base, _W)], xv)
    pltpu.sync_copy(y_hbm.at[pl.ds(base, _W)], yv)
    pltpu.sync_copy(z_hbm.at[pl.ds(base, _W)], zv)
    pltpu.sync_copy(cx_hbm, cxv)
    pltpu.sync_copy(cy_hbm, cyv)
    pltpu.sync_copy(cz_hbm, czv)
    pltpu.sync_copy(hx_hbm, hxv)
    pltpu.sync_copy(hy_hbm, hyv)
    pltpu.sync_copy(hz_hbm, hzv)

    @plsc.parallel_loop(0, _M)
    def norm_step(g):
        o = g * _L
        hx = hxv[pl.ds(o, _L)] * jnp.float32(0.5)
        hy = hyv[pl.ds(o, _L)] * jnp.float32(0.5)
        hz = hzv[pl.ds(o, _L)] * jnp.float32(0.5)
        n2v[pl.ds(o, _L)] = (hx * hx + hy * hy) + hz * hz

    inf16 = jnp.full((_L,), jnp.inf, jnp.float32)

    @plsc.parallel_loop(0, _W // (_L * _PAIR))
    def step(t):
        o = t * (_L * _PAIR)
        xs = [xv[pl.ds(o + _L * p, _L)] for p in range(_PAIR)]
        ys = [yv[pl.ds(o + _L * p, _L)] for p in range(_PAIR)]
        zs = [zv[pl.ds(o + _L * p, _L)] for p in range(_PAIR)]
        ms = [inf16] * _PAIR
        ns = [inf16] * _PAIR
        for j in range(_M):
            cxj = cxv[pl.ds(j * _L, _L)]
            cyj = cyv[pl.ds(j * _L, _L)]
            czj = czv[pl.ds(j * _L, _L)]
            n2j = n2v[pl.ds(j * _L, _L)]
            for p in range(_PAIR):
                dx = xs[p] - cxj
                dy = ys[p] - cyj
                dz = zs[p] - czj
                d2 = (dx * dx + dy * dy) + dz * dz
                lt = d2 < ms[p]
                ms[p] = jnp.where(lt, d2, ms[p])
                ns[p] = jnp.where(lt, n2j, ns[p])
        for p in range(_PAIR):
            mv[pl.ds(o + _L * p, _L)] = ms[p]
            nv[pl.ds(o + _L * p, _L)] = ns[p]

    pltpu.sync_copy(mv, mind_hbm.at[pl.ds(base, _W)])
    pltpu.sync_copy(nv, nsel_hbm.at[pl.ds(base, _W)])


def _sc_call(x, y, z, cx, cy, cz, hx, hy, hz):
    mesh = plsc.VectorSubcoreMesh(core_axis_name="c", subcore_axis_name="s")
    f = pl.kernel(
        _sc_core,
        out_type=[
            jax.ShapeDtypeStruct((_NPAD,), jnp.float32),
            jax.ShapeDtypeStruct((_NPAD,), jnp.float32),
        ],
        mesh=mesh,
        scratch_types=[
            pltpu.VMEM((_W,), jnp.float32),
            pltpu.VMEM((_W,), jnp.float32),
            pltpu.VMEM((_W,), jnp.float32),
            pltpu.VMEM((_W,), jnp.float32),
            pltpu.VMEM((_W,), jnp.float32),
            pltpu.VMEM((_MR,), jnp.float32),
            pltpu.VMEM((_MR,), jnp.float32),
            pltpu.VMEM((_MR,), jnp.float32),
            pltpu.VMEM((_MR,), jnp.float32),
            pltpu.VMEM((_MR,), jnp.float32),
            pltpu.VMEM((_MR,), jnp.float32),
            pltpu.VMEM((_MR,), jnp.float32),
        ],
    )
    return f(x, y, z, cx, cy, cz, hx, hy, hz)


def _tc_finish_body(rad_ref, pts_ref, mind2_ref, n2sel_ref,
                    sampled_ref, mind_ref, mask_ref):
    min_dis = jnp.sqrt(mind2_ref[:, :] + jnp.float32(1e-12))
    thresh = jnp.sqrt(n2sel_ref[:, :]) + rad_ref[0]
    mask = min_dis < thresh
    mind_ref[:, :] = min_dis
    mask_ref[:, :] = mask
    sampled_ref[:, :] = jnp.where(mask, pts_ref[:, :], jnp.float32(0.0))


def _tc_finish(rad, pts_t, mind2, n2sel):
    grid = _NPAD // _BN
    return pl.pallas_call(
        _tc_finish_body,
        grid=(grid,),
        in_specs=[
            pl.BlockSpec(memory_space=pltpu.SMEM),
            pl.BlockSpec((3, _BN), lambda i: (0, i)),
            pl.BlockSpec((1, _BN), lambda i: (0, i)),
            pl.BlockSpec((1, _BN), lambda i: (0, i)),
        ],
        out_specs=[
            pl.BlockSpec((3, _BN), lambda i: (0, i)),
            pl.BlockSpec((1, _BN), lambda i: (0, i)),
            pl.BlockSpec((1, _BN), lambda i: (0, i)),
        ],
        out_shape=[
            jax.ShapeDtypeStruct((3, _NPAD), jnp.float32),
            jax.ShapeDtypeStruct((1, _NPAD), jnp.float32),
            jax.ShapeDtypeStruct((1, _NPAD), jnp.bool_),
        ],
    )(rad, pts_t, mind2, n2sel)


@jax.jit
def _run(points, rois, rad):
    n = points.shape[0]
    pts_t = jnp.pad(points.T, ((0, 0), (0, _NPAD - n)))
    rep = jnp.repeat(rois, _L, axis=0)

    mind2, n2sel = _sc_call(
        pts_t[0], pts_t[1], pts_t[2],
        rep[:, 0], rep[:, 1], rep[:, 2],
        rep[:, 3], rep[:, 4], rep[:, 5])

    sampled_t, mind, mask = _tc_finish(
        rad, pts_t, mind2.reshape(1, _NPAD), n2sel.reshape(1, _NPAD))
    return (sampled_t[:, :n].T, mind[0, :n], mask[0, :n])


def kernel(points, rois, sample_radius_with_roi):
    rad = jnp.float32(sample_radius_with_roi).reshape((1,))
    return _run(points, rois, rad)


# speedup vs baseline: 1.7734x; 1.7734x over previous
import jax
import jax.numpy as jnp
import numpy as np
from jax import lax
from jax.experimental import pallas as pl
from jax.experimental.pallas import tpu as pltpu
from jax.experimental.pallas import tpu_sc as plsc

_M = 128
_BN = 3584
_NBLK = 28
_NPAD = _NBLK * _BN
_TCBLK = 20
_NT = _TCBLK * _BN
_NS = _NPAD - _NT
_NW = 32
_W = _NS // _NW
_PAIR = 2
_L = 16
_MR = _M * _L
_NSLAB = _M // 8

_PERM = np.array([(p % 8) * 16 + p // 8 for p in range(_M)], dtype=np.int32)


def _sc_core(x_hbm, y_hbm, z_hbm, cx_hbm, cy_hbm, cz_hbm,
             hx_hbm, hy_hbm, hz_hbm,
             mind_hbm, nsel_hbm,
             xv, yv, zv, mv, nv, cxv, cyv, czv, n2v, hxv, hyv, hzv):
    wid = lax.axis_index("s") * 2 + lax.axis_index("c")
    base = _NT + wid * _W
    pltpu.sync_copy(x_hbm.at[pl.ds(base, _W)], xv)
    pltpu.sync_copy(y_hbm.at[pl.ds(base, _W)], yv)
    pltpu.sync_copy(z_hbm.at[pl.ds(base, _W)], zv)
    pltpu.sync_copy(cx_hbm, cxv)
    pltpu.sync_copy(cy_hbm, cyv)
    pltpu.sync_copy(cz_hbm, czv)
    pltpu.sync_copy(hx_hbm, hxv)
    pltpu.sync_copy(hy_hbm, hyv)
    pltpu.sync_copy(hz_hbm, hzv)

    @plsc.parallel_loop(0, _M)
    def norm_step(g):
        o = g * _L
        hx = hxv[pl.ds(o, _L)] * jnp.float32(0.5)
        hy = hyv[pl.ds(o, _L)] * jnp.float32(0.5)
        hz = hzv[pl.ds(o, _L)] * jnp.float32(0.5)
        n2v[pl.ds(o, _L)] = (hx * hx + hy * hy) + hz * hz

    inf16 = jnp.full((_L,), jnp.inf, jnp.float32)

    @plsc.parallel_loop(0, _W // (_L * _PAIR))
    def step(t):
        o = t * (_L * _PAIR)
        xs = [xv[pl.ds(o + _L * p, _L)] for p in range(_PAIR)]
        ys = [yv[pl.ds(o + _L * p, _L)] for p in range(_PAIR)]
        zs = [zv[pl.ds(o + _L * p, _L)] for p in range(_PAIR)]
        ms = [inf16] * _PAIR
        ns = [inf16] * _PAIR
        for j in range(_M):
            cxj = cxv[pl.ds(j * _L, _L)]
            cyj = cyv[pl.ds(j * _L, _L)]
            czj = czv[pl.ds(j * _L, _L)]
            n2j = n2v[pl.ds(j * _L, _L)]
            for p in range(_PAIR):
                dx = xs[p] - cxj
                dy = ys[p] - cyj
                dz = zs[p] - czj
                d2 = (dx * dx + dy * dy) + dz * dz
                lt = d2 < ms[p]
                ms[p] = jnp.where(lt, d2, ms[p])
                ns[p] = jnp.where(lt, n2j, ns[p])
        for p in range(_PAIR):
            mv[pl.ds(o + _L * p, _L)] = ms[p]
            nv[pl.ds(o + _L * p, _L)] = ns[p]

    pltpu.sync_copy(mv, mind_hbm.at[pl.ds(wid * _W, _W)])
    pltpu.sync_copy(nv, nsel_hbm.at[pl.ds(wid * _W, _W)])


def _sc_call(x, y, z, cx, cy, cz, hx, hy, hz):
    mesh = plsc.VectorSubcoreMesh(core_axis_name="c", subcore_axis_name="s")
    f = pl.kernel(
        _sc_core,
        out_type=[
            jax.ShapeDtypeStruct((_NS,), jnp.float32),
            jax.ShapeDtypeStruct((_NS,), jnp.float32),
        ],
        mesh=mesh,
        scratch_types=[
            pltpu.VMEM((_W,), jnp.float32),
            pltpu.VMEM((_W,), jnp.float32),
            pltpu.VMEM((_W,), jnp.float32),
            pltpu.VMEM((_W,), jnp.float32),
            pltpu.VMEM((_W,), jnp.float32),
            pltpu.VMEM((_MR,), jnp.float32),
            pltpu.VMEM((_MR,), jnp.float32),
            pltpu.VMEM((_MR,), jnp.float32),
            pltpu.VMEM((_MR,), jnp.float32),
            pltpu.VMEM((_MR,), jnp.float32),
            pltpu.VMEM((_MR,), jnp.float32),
            pltpu.VMEM((_MR,), jnp.float32),
        ],
    )
    return f(x, y, z, cx, cy, cz, hx, hy, hz)


def _tc_finish_body(rad_ref, pts_ref, mind2_ref, n2sel_ref,
                    sampled_ref, mind_ref, mask_ref):
    min_dis = jnp.sqrt(mind2_ref[:, :] + jnp.float32(1e-12))
    thresh = jnp.sqrt(n2sel_ref[:, :]) + rad_ref[0]
    mask = min_dis < thresh
    mind_ref[:, :] = min_dis
    mask_ref[:, :] = mask
    sampled_ref[:, :] = jnp.where(mask, pts_ref[:, :], jnp.float32(0.0))


def _tc_finish(rad, pts_t, mind2, n2sel):
    grid = _NS // _BN
    return pl.pallas_call(
        _tc_finish_body,
        grid=(grid,),
        in_specs=[
            pl.BlockSpec(memory_space=pltpu.SMEM),
            pl.BlockSpec((3, _BN), lambda i: (0, i + _TCBLK)),
            pl.BlockSpec((1, _BN), lambda i: (0, i)),
            pl.BlockSpec((1, _BN), lambda i: (0, i)),
        ],
        out_specs=[
            pl.BlockSpec((3, _BN), lambda i: (0, i)),
            pl.BlockSpec((1, _BN), lambda i: (0, i)),
            pl.BlockSpec((1, _BN), lambda i: (0, i)),
        ],
        out_shape=[
            jax.ShapeDtypeStruct((3, _NS), jnp.float32),
            jax.ShapeDtypeStruct((1, _NS), jnp.float32),
            jax.ShapeDtypeStruct((1, _NS), jnp.bool_),
        ],
    )(rad, pts_t, mind2, n2sel)


def _tc_body(rad_ref, pts_ref, rois_ref, sampled_ref, mind_ref, mask_ref):
    hx = rois_ref[:, 3:4] * jnp.float32(0.5)
    hy = rois_ref[:, 4:5] * jnp.float32(0.5)
    hz = rois_ref[:, 5:6] * jnp.float32(0.5)
    rnorm = jnp.sqrt((hx * hx + hy * hy) + hz * hz)
    rad = rad_ref[0]

    nchain = 2
    per = _NSLAB // nchain
    BLK = 512
    for col in range(_BN // BLK):
        sl = slice(col * BLK, (col + 1) * BLK)
        px = pts_ref[0:1, sl]
        py = pts_ref[1:2, sl]
        pz = pts_ref[2:3, sl]

        ms = [None] * nchain
        tvs = [None] * nchain
        for c in range(nchain):
            for k in range(per):
                i = c * per + k
                cx = rois_ref[8 * i:8 * i + 8, 0:1]
                cy = rois_ref[8 * i:8 * i + 8, 1:2]
                cz = rois_ref[8 * i:8 * i + 8, 2:3]
                dx = px - cx
                dy = py - cy
                dz = pz - cz
                d2 = (dx * dx + dy * dy) + dz * dz
                rn = rnorm[8 * i:8 * i + 8, 0:1]
                if k == 0:
                    ms[c] = d2
                    tvs[c] = jnp.broadcast_to(rn, d2.shape)
                else:
                    lt = d2 < ms[c]
                    ms[c] = jnp.where(lt, d2, ms[c])
                    tvs[c] = jnp.where(lt, rn, tvs[c])
        while len(ms) > 1:
            nms, ntvs = [], []
            for c in range(0, len(ms), 2):
                lt = ms[c + 1] < ms[c]
                nms.append(jnp.where(lt, ms[c + 1], ms[c]))
                ntvs.append(jnp.where(lt, tvs[c + 1], tvs[c]))
            ms, tvs = nms, ntvs
        m, tv = ms[0], tvs[0]

        for h in (4, 2, 1):
            lt = m[h:2 * h, :] < m[:h, :]
            m = jnp.where(lt, m[h:2 * h, :], m[:h, :])
            tv = jnp.where(lt, tv[h:2 * h, :], tv[:h, :])

        min_dis = jnp.sqrt(m + jnp.float32(1e-12))
        mask = min_dis < tv + rad

        mind_ref[:, sl] = min_dis
        mask_ref[:, sl] = mask
        sampled_ref[:, sl] = jnp.where(
            mask, pts_ref[:, sl], jnp.float32(0.0))


def _tc_call(rad, pts_t, rois_k):
    return pl.pallas_call(
        _tc_body,
        grid=(_TCBLK,),
        in_specs=[
            pl.BlockSpec(memory_space=pltpu.SMEM),
            pl.BlockSpec((3, _BN), lambda i: (0, i)),
            pl.BlockSpec((_M, 7), lambda i: (0, 0)),
        ],
        out_specs=[
            pl.BlockSpec((3, _BN), lambda i: (0, i)),
            pl.BlockSpec((1, _BN), lambda i: (0, i)),
            pl.BlockSpec((1, _BN), lambda i: (0, i)),
        ],
        out_shape=[
            jax.ShapeDtypeStruct((3, _NT), jnp.float32),
            jax.ShapeDtypeStruct((1, _NT), jnp.float32),
            jax.ShapeDtypeStruct((1, _NT), jnp.bool_),
        ],
    )(rad, pts_t, rois_k)


@jax.jit
def _run(points, rois, rad):
    n = points.shape[0]
    pts_t = jnp.pad(points.T, ((0, 0), (0, _NPAD - n)))
    rois_k = rois[_PERM]
    rep = jnp.repeat(rois, _L, axis=0)

    mind2_sc, n2sel_sc = _sc_call(
        pts_t[0], pts_t[1], pts_t[2],
        rep[:, 0], rep[:, 1], rep[:, 2],
        rep[:, 3], rep[:, 4], rep[:, 5])

    s_tc, d_tc, k_tc = _tc_call(rad, pts_t, rois_k)

    s_sc, d_sc, k_sc = _tc_finish(
        rad, pts_t, mind2_sc.reshape(1, _NS), n2sel_sc.reshape(1, _NS))

    sampled_t = jnp.concatenate([s_tc, s_sc], axis=1)
    mind = jnp.concatenate([d_tc, d_sc], axis=1)
    mask = jnp.concatenate([k_tc, k_sc], axis=1)
    return (sampled_t[:, :n].T, mind[0, :n], mask[0, :n])


def kernel(points, rois, sample_radius_with_roi):
    rad = jnp.float32(sample_radius_with_roi).reshape((1,))
    return _run(points, rois, rad)


# speedup vs baseline: 2.5874x; 1.4590x over previous
import jax
import jax.numpy as jnp
import numpy as np
from jax.experimental import pallas as pl
from jax.experimental.pallas import tpu as pltpu

_M = 128
_BN = 3584
_NPAD = 100352
_NSLAB = _M // 8

_PERM = np.array([(p % 8) * 16 + p // 8 for p in range(_M)], dtype=np.int32)


def _body(rad_ref, pts_ref, rois_ref, sampled_ref, mind_ref, mask_ref):
    hx = rois_ref[:, 3:4] * jnp.float32(0.5)
    hy = rois_ref[:, 4:5] * jnp.float32(0.5)
    hz = rois_ref[:, 5:6] * jnp.float32(0.5)
    rnorm = jnp.sqrt((hx * hx + hy * hy) + hz * hz)
    rad = rad_ref[0]

    nchain = 2
    per = _NSLAB // nchain
    BLK = 512
    for col in range(_BN // BLK):
        sl = slice(col * BLK, (col + 1) * BLK)
        px = pts_ref[0:1, sl]
        py = pts_ref[1:2, sl]
        pz = pts_ref[2:3, sl]

        ms = [None] * nchain
        tvs = [None] * nchain
        for c in range(nchain):
            for k in range(per):
                i = c * per + k
                cx = rois_ref[8 * i:8 * i + 8, 0:1]
                cy = rois_ref[8 * i:8 * i + 8, 1:2]
                cz = rois_ref[8 * i:8 * i + 8, 2:3]
                dx = px - cx
                dy = py - cy
                dz = pz - cz
                d2 = (dx * dx + dy * dy) + dz * dz
                rn = rnorm[8 * i:8 * i + 8, 0:1]
                if k == 0:
                    ms[c] = d2
                    tvs[c] = jnp.broadcast_to(rn, d2.shape)
                else:
                    lt = d2 < ms[c]
                    ms[c] = jnp.where(lt, d2, ms[c])
                    tvs[c] = jnp.where(lt, rn, tvs[c])
        while len(ms) > 1:
            nms, ntvs = [], []
            for c in range(0, len(ms), 2):
                lt = ms[c + 1] < ms[c]
                nms.append(jnp.where(lt, ms[c + 1], ms[c]))
                ntvs.append(jnp.where(lt, tvs[c + 1], tvs[c]))
            ms, tvs = nms, ntvs
        m, tv = ms[0], tvs[0]

        for h in (4, 2, 1):
            lt = m[h:2 * h, :] < m[:h, :]
            m = jnp.where(lt, m[h:2 * h, :], m[:h, :])
            tv = jnp.where(lt, tv[h:2 * h, :], tv[:h, :])

        min_dis = jnp.sqrt(m + jnp.float32(1e-12))
        mask = min_dis < tv + rad

        mind_ref[:, sl] = min_dis
        mask_ref[:, sl] = mask
        sampled_ref[:, sl] = jnp.where(
            mask, pts_ref[:, sl], jnp.float32(0.0))


@jax.jit
def _run(points, rois, rad):
    n = points.shape[0]
    pts_t = jnp.pad(points.T, ((0, 0), (0, _NPAD - n)))
    rois_k = rois[_PERM]
    grid = _NPAD // _BN

    sampled_t, mind, mask = pl.pallas_call(
        _body,
        grid=(grid,),
        in_specs=[
            pl.BlockSpec(memory_space=pltpu.SMEM),
            pl.BlockSpec((3, _BN), lambda i: (0, i)),
            pl.BlockSpec((_M, 7), lambda i: (0, 0)),
        ],
        out_specs=[
            pl.BlockSpec((3, _BN), lambda i: (0, i)),
            pl.BlockSpec((1, _BN), lambda i: (0, i)),
            pl.BlockSpec((1, _BN), lambda i: (0, i)),
        ],
        out_shape=[
            jax.ShapeDtypeStruct((3, _NPAD), jnp.float32),
            jax.ShapeDtypeStruct((1, _NPAD), jnp.float32),
            jax.ShapeDtypeStruct((1, _NPAD), jnp.bool_),
        ],
    )(rad, pts_t, rois_k)
    return (sampled_t[:, :n].T, mind[0, :n], mask[0, :n])


def kernel(points, rois, sample_radius_with_roi):
    rad = jnp.float32(sample_radius_with_roi).reshape((1,))
    return _run(points, rois, rad)
